# Initial kernel scaffold; baseline (speedup 1.0000x reference)
import jax
import jax.numpy as jnp
from jax.experimental import pallas as pl


def kernel(x, edge_index, batch, emb, W1, b1, g1, be1, W2, b2, g2, be2, fcW, fcb):
    raise NotImplementedError("write your pallas kernel here")


# speedup vs baseline: 12.2426x; 12.2426x over previous
import functools

import jax
import jax.numpy as jnp
from jax import lax
from jax.experimental import pallas as pl
from jax.experimental.pallas import tpu as pltpu
import jax.experimental.pallas.tpu_sc as plsc

N_NODES = 10000
N_EDGES = 320000
N_GRAPHS = 64
D = 128

NC, NS = 2, 16
NW = NC * NS
NPAD = 10240
ROWS_PW = NPAD // NW
EDGES_PW = N_EDGES // NW
ECHUNK = 80
NCHUNK = EDGES_PW // ECHUNK
ROWS_PT = NPAD // NS

_BLK = 1024
_NBLK = NPAD // _BLK


def _mesh():
    return plsc.VectorSubcoreMesh(core_axis_name="c", subcore_axis_name="s")


def _sc_gather_deg(xp, dst, emb):
    @functools.partial(
        pl.kernel,
        mesh=_mesh(),
        out_type=[
            jax.ShapeDtypeStruct((NPAD, D), jnp.float32),
            jax.ShapeDtypeStruct((NC, NPAD), jnp.float32),
        ],
        scratch_types=[
            pltpu.VMEM((ECHUNK,), jnp.int32),
            pltpu.VMEM((ECHUNK,), jnp.int32),
            pltpu.VMEM((ECHUNK,), jnp.float32),
            pltpu.VMEM((ROWS_PT,), jnp.float32),
            pltpu.VMEM((ECHUNK, D), jnp.float32),
            pltpu.VMEM_SHARED((NPAD,), jnp.float32),
            pltpu.SemaphoreType.DMA,
        ],
    )
    def body(x_hbm, dst_hbm, emb_hbm, h_out, deg_out, idx_v, dst_v, ones_v,
             zbuf, rows_v, shared_deg, sem):
        c = lax.axis_index("c")
        s = lax.axis_index("s")
        wid = s * NC + c

        for j in range(ECHUNK // 16):
            ones_v[pl.ds(16 * j, 16)] = jnp.ones((16,), jnp.float32)
        for j in range(ROWS_PT // 16):
            zbuf[pl.ds(16 * j, 16)] = jnp.zeros((16,), jnp.float32)

        pltpu.sync_copy(zbuf, shared_deg.at[pl.ds(s * ROWS_PT, ROWS_PT)])
        plsc.subcore_barrier()

        for j in range(ROWS_PW // ECHUNK):
            base = wid * ROWS_PW + j * ECHUNK
            pltpu.sync_copy(x_hbm.at[pl.ds(base, ECHUNK)], idx_v)
            pltpu.async_copy(emb_hbm.at[idx_v], rows_v, sem).wait()
            pltpu.sync_copy(rows_v, h_out.at[pl.ds(base, ECHUNK)])

        def dbody(i, carry):
            base = wid * EDGES_PW + i * ECHUNK
            pltpu.sync_copy(dst_hbm.at[pl.ds(base, ECHUNK)], dst_v)
            pltpu.sync_copy(ones_v, shared_deg.at[dst_v], add=True)
            return carry

        lax.fori_loop(0, NCHUNK, dbody, 0)
        plsc.subcore_barrier()

        pltpu.sync_copy(shared_deg.at[pl.ds(s * ROWS_PT, ROWS_PT)], zbuf)
        pltpu.sync_copy(zbuf, deg_out.at[c, pl.ds(s * ROWS_PT, ROWS_PT)])

    return body(xp, dst, emb)


def _sc_edge_agg(hws, src, dst):
    @functools.partial(
        pl.kernel,
        mesh=_mesh(),
        out_type=jax.ShapeDtypeStruct((NC, NPAD, D), jnp.float32),
        scratch_types=[
            pltpu.VMEM((ECHUNK,), jnp.int32),
            pltpu.VMEM((ECHUNK,), jnp.int32),
            pltpu.VMEM((ECHUNK, D), jnp.float32),
            pltpu.VMEM_SHARED((NPAD, D), jnp.float32),
            pltpu.SemaphoreType.DMA,
        ],
    )
    def body(hws_hbm, src_hbm, dst_hbm, acc_out, src_v, dst_v, rows_v,
             shared_acc, sem):
        c = lax.axis_index("c")
        s = lax.axis_index("s")
        wid = s * NC + c

        for j in range(8):
            def zb(i, carry, _col=j):
                rows_v[i, pl.ds(_col * 16, 16)] = jnp.zeros((16,), jnp.float32)
                return carry
            lax.fori_loop(0, ECHUNK, zb, 0)
        for k in range(ROWS_PT // ECHUNK):
            pltpu.sync_copy(
                rows_v, shared_acc.at[pl.ds(s * ROWS_PT + k * ECHUNK, ECHUNK)])
        plsc.subcore_barrier()

        def ebody(i, carry):
            base = wid * EDGES_PW + i * ECHUNK
            pltpu.sync_copy(src_hbm.at[pl.ds(base, ECHUNK)], src_v)
            pltpu.sync_copy(dst_hbm.at[pl.ds(base, ECHUNK)], dst_v)
            pltpu.async_copy(hws_hbm.at[src_v], rows_v, sem).wait()
            pltpu.sync_copy(rows_v, shared_acc.at[dst_v], add=True)
            return carry

        lax.fori_loop(0, NCHUNK, ebody, 0)
        plsc.subcore_barrier()

        for k in range(ROWS_PT // ECHUNK):
            r0 = s * ROWS_PT + k * ECHUNK
            pltpu.sync_copy(shared_acc.at[pl.ds(r0, ECHUNK)], rows_v)
            pltpu.sync_copy(rows_v, acc_out.at[c, pl.ds(r0, ECHUNK)])

    return body(hws, src, dst)


def _tc_prep1(deg_t, h, W1):
    def body(degt_ref, h_ref, w_ref, hws_ref, dinv_ref):
        deg = degt_ref[:, 0:1] + degt_ref[:, 1:2] + 1.0
        dinv = lax.rsqrt(deg)
        dinv_ref[...] = dinv
        hw = jnp.dot(h_ref[...], w_ref[...], preferred_element_type=jnp.float32)
        hws_ref[...] = hw * dinv

    return pl.pallas_call(
        body,
        grid=(_NBLK,),
        in_specs=[
            pl.BlockSpec((_BLK, 2), lambda i: (i, 0)),
            pl.BlockSpec((_BLK, D), lambda i: (i, 0)),
            pl.BlockSpec((D, D), lambda i: (0, 0)),
        ],
        out_specs=[
            pl.BlockSpec((_BLK, D), lambda i: (i, 0)),
            pl.BlockSpec((_BLK, 1), lambda i: (i, 0)),
        ],
        out_shape=[
            jax.ShapeDtypeStruct((NPAD, D), jnp.float32),
            jax.ShapeDtypeStruct((NPAD, 1), jnp.float32),
        ],
    )(deg_t, h, W1)


def _norm_act(accp, hws, dinv, b, g, be):
    acc = accp[0] + accp[1]
    pre = dinv * (acc + hws) + b
    mu = jnp.mean(pre, axis=1, keepdims=True)
    yc = pre - mu
    var = jnp.mean(yc * yc, axis=1, keepdims=True)
    hn = yc * lax.rsqrt(var + 1e-5) * g + be
    return jnp.maximum(hn, 0.0)


def _tc_combine1(accp, hws1, dinv, b1, g1, be1, W2):
    def body(accp_ref, hws_ref, dinv_ref, b_ref, g_ref, be_ref, w_ref, out_ref):
        h1 = _norm_act(accp_ref[...], hws_ref[...], dinv_ref[...],
                       b_ref[...], g_ref[...], be_ref[...])
        hw = jnp.dot(h1, w_ref[...], preferred_element_type=jnp.float32)
        out_ref[...] = hw * dinv_ref[...]

    return pl.pallas_call(
        body,
        grid=(_NBLK,),
        in_specs=[
            pl.BlockSpec((NC, _BLK, D), lambda i: (0, i, 0)),
            pl.BlockSpec((_BLK, D), lambda i: (i, 0)),
            pl.BlockSpec((_BLK, 1), lambda i: (i, 0)),
            pl.BlockSpec((1, D), lambda i: (0, 0)),
            pl.BlockSpec((1, D), lambda i: (0, 0)),
            pl.BlockSpec((1, D), lambda i: (0, 0)),
            pl.BlockSpec((D, D), lambda i: (0, 0)),
        ],
        out_specs=pl.BlockSpec((_BLK, D), lambda i: (i, 0)),
        out_shape=jax.ShapeDtypeStruct((NPAD, D), jnp.float32),
    )(accp, hws1, dinv, b1.reshape(1, D), g1.reshape(1, D),
      be1.reshape(1, D), W2)


def _tc_combine2(accp, hws2, dinv, b2, g2, be2, batch_col, fcW, fcb):
    def body(accp_ref, hws_ref, dinv_ref, b_ref, g_ref, be_ref, bat_ref,
             fcw_ref, fcb_ref, out_ref, sums_sc, cnts_sc):
        h2 = _norm_act(accp_ref[...], hws_ref[...], dinv_ref[...],
                       b_ref[...], g_ref[...], be_ref[...])
        bid = bat_ref[...]
        gi = lax.broadcasted_iota(jnp.int32, (_BLK, N_GRAPHS), 1)
        oh = (bid == gi).astype(jnp.float32)
        dn = (((0,), (0,)), ((), ()))
        ps = lax.dot_general(oh, h2, dn, preferred_element_type=jnp.float32)
        ones_m = jnp.ones((_BLK, D), jnp.float32)
        pc = lax.dot_general(oh, ones_m, dn, preferred_element_type=jnp.float32)

        i = pl.program_id(0)

        @pl.when(i == 0)
        def _():
            sums_sc[...] = ps
            cnts_sc[...] = pc
            out_ref[...] = jnp.zeros((N_GRAPHS, 1), jnp.float32)

        @pl.when(i > 0)
        def _():
            sums_sc[...] += ps
            cnts_sc[...] += pc

        @pl.when(i == _NBLK - 1)
        def _():
            pooled = sums_sc[...] / jnp.maximum(cnts_sc[...], 1.0)
            out_ref[...] = (jnp.dot(pooled, fcw_ref[...],
                                    preferred_element_type=jnp.float32)
                            + fcb_ref[...])

    return pl.pallas_call(
        body,
        grid=(_NBLK,),
        in_specs=[
            pl.BlockSpec((NC, _BLK, D), lambda i: (0, i, 0)),
            pl.BlockSpec((_BLK, D), lambda i: (i, 0)),
            pl.BlockSpec((_BLK, 1), lambda i: (i, 0)),
            pl.BlockSpec((1, D), lambda i: (0, 0)),
            pl.BlockSpec((1, D), lambda i: (0, 0)),
            pl.BlockSpec((1, D), lambda i: (0, 0)),
            pl.BlockSpec((_BLK, 1), lambda i: (i, 0)),
            pl.BlockSpec((D, 1), lambda i: (0, 0)),
            pl.BlockSpec((1, 1), lambda i: (0, 0)),
        ],
        out_specs=pl.BlockSpec((N_GRAPHS, 1), lambda i: (0, 0)),
        out_shape=jax.ShapeDtypeStruct((N_GRAPHS, 1), jnp.float32),
        scratch_shapes=[
            pltpu.VMEM((N_GRAPHS, D), jnp.float32),
            pltpu.VMEM((N_GRAPHS, D), jnp.float32),
        ],
    )(accp, hws2, dinv, b2.reshape(1, D), g2.reshape(1, D), be2.reshape(1, D),
      batch_col, fcW, fcb.reshape(1, 1))


def kernel(x, edge_index, batch, emb, W1, b1, g1, be1, W2, b2, g2, be2,
           fcW, fcb):
    x = x.astype(jnp.int32)
    src = edge_index[0].astype(jnp.int32)
    dst = edge_index[1].astype(jnp.int32)
    xp = jnp.concatenate([x, jnp.arange(NPAD - N_NODES, dtype=jnp.int32)])
    batch_col = jnp.concatenate(
        [batch.astype(jnp.int32),
         jnp.full((NPAD - N_NODES,), N_GRAPHS, jnp.int32)]).reshape(NPAD, 1)

    h, degp = _sc_gather_deg(xp, dst, emb)
    hws1, dinv = _tc_prep1(degp.T, h, W1)
    accp1 = _sc_edge_agg(hws1, src, dst)
    hws2 = _tc_combine1(accp1, hws1, dinv, b1, g1, be1, W2)
    accp2 = _sc_edge_agg(hws2, src, dst)
    out_col = _tc_combine2(accp2, hws2, dinv, b2, g2, be2, batch_col, fcW, fcb)
    return out_col[:, 0]


# speedup vs baseline: 26.0506x; 2.1279x over previous
import functools

import jax
import jax.numpy as jnp
from jax import lax
from jax.experimental import pallas as pl
from jax.experimental.pallas import tpu as pltpu
import jax.experimental.pallas.tpu_sc as plsc

N_NODES = 10000
N_EDGES = 320000
N_GRAPHS = 64
D = 128

NC, NS = 2, 16
NW = NC * NS
NPAD = 10240
ROWS_PW = NPAD // NW
ECHUNK = 128
NCHUNK = 80
EPAD = NW * NCHUNK * ECHUNK
EDGES_PW = NCHUNK * ECHUNK
GCHUNK = 80
ROWS_PT = NPAD // NS

_BLK = 1024
_NBLK = NPAD // _BLK


def _mesh():
    return plsc.VectorSubcoreMesh(core_axis_name="c", subcore_axis_name="s")


def _sc_gather_deg(xp, dst3, emb):
    @functools.partial(
        pl.kernel,
        mesh=_mesh(),
        out_type=[
            jax.ShapeDtypeStruct((NPAD, D), jnp.float32),
            jax.ShapeDtypeStruct((NC, NPAD), jnp.float32),
        ],
        scratch_types=[
            pltpu.VMEM((ROWS_PW,), jnp.int32),
            pltpu.VMEM((NCHUNK, ECHUNK), jnp.int32),
            pltpu.VMEM((ECHUNK,), jnp.float32),
            pltpu.VMEM((ROWS_PT,), jnp.float32),
            pltpu.VMEM((GCHUNK, D), jnp.float32),
            pltpu.VMEM((GCHUNK, D), jnp.float32),
            pltpu.VMEM_SHARED((NPAD,), jnp.float32),
            pltpu.SemaphoreType.DMA,
            pltpu.SemaphoreType.DMA,
        ],
    )
    def body(x_hbm, dst_hbm, emb_hbm, h_out, deg_out, idx4, dst2d,
             ones_v, zbuf, rows_a, rows_b, shared_deg, sem_g, sem_h):
        c = lax.axis_index("c")
        s = lax.axis_index("s")
        wid = s * NC + c

        for j in range(ECHUNK // 16):
            ones_v[pl.ds(16 * j, 16)] = jnp.ones((16,), jnp.float32)
        for j in range(ROWS_PT // 16):
            zbuf[pl.ds(16 * j, 16)] = jnp.zeros((16,), jnp.float32)

        pltpu.sync_copy(zbuf, shared_deg.at[pl.ds(s * ROWS_PT, ROWS_PT)])
        pltpu.sync_copy(dst_hbm.at[wid], dst2d)
        plsc.subcore_barrier()

        nge = ROWS_PW // GCHUNK
        pltpu.sync_copy(x_hbm.at[pl.ds(wid * ROWS_PW, ROWS_PW)], idx4)
        ids = [idx4.at[pl.ds(j * GCHUNK, GCHUNK)] for j in range(nge)]
        rows = [rows_a, rows_b]
        pltpu.async_copy(emb_hbm.at[ids[0]], rows_a, sem_g)
        for j in range(nge):
            pltpu.make_async_copy(emb_hbm.at[ids[j]], rows[j % 2],
                                  sem_g).wait()
            if j + 1 < nge:
                pltpu.async_copy(emb_hbm.at[ids[j + 1]],
                                 rows[(j + 1) % 2], sem_g)
            pltpu.sync_copy(rows[j % 2],
                            h_out.at[pl.ds(wid * ROWS_PW + j * GCHUNK,
                                           GCHUNK)])

        def hgroup(g, carry):
            for j in range(8):
                pltpu.async_copy(ones_v, shared_deg.at[dst2d.at[g * 8 + j]],
                                 sem_h, add=True)
            for j in range(8):
                pltpu.make_async_copy(ones_v,
                                      shared_deg.at[dst2d.at[g * 8 + j]],
                                      sem_h).wait()
            return carry

        lax.fori_loop(0, NCHUNK // 8, hgroup, 0)
        plsc.subcore_barrier()

        pltpu.sync_copy(shared_deg.at[pl.ds(s * ROWS_PT, ROWS_PT)], zbuf)
        pltpu.sync_copy(zbuf, deg_out.at[c, pl.ds(s * ROWS_PT, ROWS_PT)])

    return body(xp, dst3, emb)


def _sc_edge_agg(hws, src3, dst3):
    @functools.partial(
        pl.kernel,
        mesh=_mesh(),
        out_type=jax.ShapeDtypeStruct((NC, NPAD, D), jnp.float32),
        scratch_types=[
            pltpu.VMEM((NCHUNK // 2, ECHUNK), jnp.int32),
            pltpu.VMEM((NCHUNK // 2, ECHUNK), jnp.int32),
            pltpu.VMEM((ECHUNK, D), jnp.float32),
            pltpu.VMEM((ECHUNK, D), jnp.float32),
            pltpu.VMEM_SHARED((NPAD, D), jnp.float32),
            pltpu.SemaphoreType.DMA,
        ],
    )
    def body(hws_hbm, src_hbm, dst_hbm, acc_out, src2d, dst2d, rows_a,
             rows_b, shared_acc, sem_g):
        c = lax.axis_index("c")
        s = lax.axis_index("s")
        wid = s * NC + c
        cph = NCHUNK // 2

        for j in range(8):
            def zb(i, carry, _col=j):
                rows_a[i, pl.ds(_col * 16, 16)] = jnp.zeros((16,), jnp.float32)
                return carry
            lax.fori_loop(0, ECHUNK, zb, 0)
        for k in range(ROWS_PT // ECHUNK):
            pltpu.sync_copy(
                rows_a, shared_acc.at[pl.ds(s * ROWS_PT + k * ECHUNK, ECHUNK)])
        plsc.subcore_barrier()

        rows = [rows_a, rows_b]
        for ph in range(2):
            pltpu.sync_copy(src_hbm.at[wid, pl.ds(ph * cph, cph)], src2d)
            pltpu.sync_copy(dst_hbm.at[wid, pl.ds(ph * cph, cph)], dst2d)
            pltpu.async_copy(hws_hbm.at[src2d.at[0]], rows_a, sem_g)

            def pair(k, carry):
                i0 = 2 * k
                for j in range(2):
                    pltpu.make_async_copy(hws_hbm.at[src2d.at[i0 + j]],
                                          rows[j % 2], sem_g).wait()

                    @pl.when(i0 + j + 1 < cph)
                    def _():
                        pltpu.async_copy(hws_hbm.at[src2d.at[i0 + j + 1]],
                                         rows[(j + 1) % 2], sem_g)

                    pltpu.sync_copy(rows[j % 2],
                                    shared_acc.at[dst2d.at[i0 + j]],
                                    add=True)
                return carry

            lax.fori_loop(0, cph // 2, pair, 0)
        plsc.subcore_barrier()

        for k in range(ROWS_PT // ECHUNK):
            r0 = s * ROWS_PT + k * ECHUNK
            pltpu.sync_copy(shared_acc.at[pl.ds(r0, ECHUNK)], rows_a)
            pltpu.sync_copy(rows_a, acc_out.at[c, pl.ds(r0, ECHUNK)])

    return body(hws, src3, dst3)


def _tc_prep1(deg_t, h, W1):
    def body(degt_ref, h_ref, w_ref, hws_ref, dinv_ref):
        deg = degt_ref[:, 0:1] + degt_ref[:, 1:2] + 1.0
        dinv = lax.rsqrt(deg)
        dinv_ref[...] = dinv
        hw = jnp.dot(h_ref[...], w_ref[...], preferred_element_type=jnp.float32)
        hws_ref[...] = hw * dinv

    return pl.pallas_call(
        body,
        grid=(_NBLK,),
        in_specs=[
            pl.BlockSpec((_BLK, 2), lambda i: (i, 0)),
            pl.BlockSpec((_BLK, D), lambda i: (i, 0)),
            pl.BlockSpec((D, D), lambda i: (0, 0)),
        ],
        out_specs=[
            pl.BlockSpec((_BLK, D), lambda i: (i, 0)),
            pl.BlockSpec((_BLK, 1), lambda i: (i, 0)),
        ],
        out_shape=[
            jax.ShapeDtypeStruct((NPAD, D), jnp.float32),
            jax.ShapeDtypeStruct((NPAD, 1), jnp.float32),
        ],
    )(deg_t, h, W1)


def _norm_act(accp, hws, dinv, b, g, be):
    acc = accp[0] + accp[1]
    pre = dinv * (acc + hws) + b
    mu = jnp.mean(pre, axis=1, keepdims=True)
    yc = pre - mu
    var = jnp.mean(yc * yc, axis=1, keepdims=True)
    hn = yc * lax.rsqrt(var + 1e-5) * g + be
    return jnp.maximum(hn, 0.0)


def _tc_combine1(accp, hws1, dinv, b1, g1, be1, W2):
    def body(accp_ref, hws_ref, dinv_ref, b_ref, g_ref, be_ref, w_ref, out_ref):
        h1 = _norm_act(accp_ref[...], hws_ref[...], dinv_ref[...],
                       b_ref[...], g_ref[...], be_ref[...])
        hw = jnp.dot(h1, w_ref[...], preferred_element_type=jnp.float32)
        out_ref[...] = hw * dinv_ref[...]

    return pl.pallas_call(
        body,
        grid=(_NBLK,),
        in_specs=[
            pl.BlockSpec((NC, _BLK, D), lambda i: (0, i, 0)),
            pl.BlockSpec((_BLK, D), lambda i: (i, 0)),
            pl.BlockSpec((_BLK, 1), lambda i: (i, 0)),
            pl.BlockSpec((1, D), lambda i: (0, 0)),
            pl.BlockSpec((1, D), lambda i: (0, 0)),
            pl.BlockSpec((1, D), lambda i: (0, 0)),
            pl.BlockSpec((D, D), lambda i: (0, 0)),
        ],
        out_specs=pl.BlockSpec((_BLK, D), lambda i: (i, 0)),
        out_shape=jax.ShapeDtypeStruct((NPAD, D), jnp.float32),
    )(accp, hws1, dinv, b1.reshape(1, D), g1.reshape(1, D),
      be1.reshape(1, D), W2)


def _tc_combine2(accp, hws2, dinv, b2, g2, be2, batch_col, fcW, fcb):
    def body(accp_ref, hws_ref, dinv_ref, b_ref, g_ref, be_ref, bat_ref,
             fcw_ref, fcb_ref, out_ref, sums_sc, cnts_sc):
        h2 = _norm_act(accp_ref[...], hws_ref[...], dinv_ref[...],
                       b_ref[...], g_ref[...], be_ref[...])
        bid = bat_ref[...]
        gi = lax.broadcasted_iota(jnp.int32, (_BLK, N_GRAPHS), 1)
        oh = (bid == gi).astype(jnp.float32)
        dn = (((0,), (0,)), ((), ()))
        ps = lax.dot_general(oh, h2, dn, preferred_element_type=jnp.float32)
        ones_m = jnp.ones((_BLK, D), jnp.float32)
        pc = lax.dot_general(oh, ones_m, dn, preferred_element_type=jnp.float32)

        i = pl.program_id(0)

        @pl.when(i == 0)
        def _():
            sums_sc[...] = ps
            cnts_sc[...] = pc
            out_ref[...] = jnp.zeros((N_GRAPHS, 1), jnp.float32)

        @pl.when(i > 0)
        def _():
            sums_sc[...] += ps
            cnts_sc[...] += pc

        @pl.when(i == _NBLK - 1)
        def _():
            pooled = sums_sc[...] / jnp.maximum(cnts_sc[...], 1.0)
            out_ref[...] = (jnp.dot(pooled, fcw_ref[...],
                                    preferred_element_type=jnp.float32)
                            + fcb_ref[...])

    return pl.pallas_call(
        body,
        grid=(_NBLK,),
        in_specs=[
            pl.BlockSpec((NC, _BLK, D), lambda i: (0, i, 0)),
            pl.BlockSpec((_BLK, D), lambda i: (i, 0)),
            pl.BlockSpec((_BLK, 1), lambda i: (i, 0)),
            pl.BlockSpec((1, D), lambda i: (0, 0)),
            pl.BlockSpec((1, D), lambda i: (0, 0)),
            pl.BlockSpec((1, D), lambda i: (0, 0)),
            pl.BlockSpec((_BLK, 1), lambda i: (i, 0)),
            pl.BlockSpec((D, 1), lambda i: (0, 0)),
            pl.BlockSpec((1, 1), lambda i: (0, 0)),
        ],
        out_specs=pl.BlockSpec((N_GRAPHS, 1), lambda i: (0, 0)),
        out_shape=jax.ShapeDtypeStruct((N_GRAPHS, 1), jnp.float32),
        scratch_shapes=[
            pltpu.VMEM((N_GRAPHS, D), jnp.float32),
            pltpu.VMEM((N_GRAPHS, D), jnp.float32),
        ],
    )(accp, hws2, dinv, b2.reshape(1, D), g2.reshape(1, D), be2.reshape(1, D),
      batch_col, fcW, fcb.reshape(1, 1))


def kernel(x, edge_index, batch, emb, W1, b1, g1, be1, W2, b2, g2, be2,
           fcW, fcb):
    x = x.astype(jnp.int32)
    src = edge_index[0].astype(jnp.int32)
    dst = edge_index[1].astype(jnp.int32)
    xp = jnp.concatenate([x, jnp.arange(NPAD - N_NODES, dtype=jnp.int32)])
    batch_col = jnp.concatenate(
        [batch.astype(jnp.int32),
         jnp.full((NPAD - N_NODES,), N_GRAPHS, jnp.int32)]).reshape(NPAD, 1)
    npe = EPAD - N_EDGES
    pad_src = jnp.arange(npe, dtype=jnp.int32) % N_NODES
    pad_dst = N_NODES + jnp.arange(npe, dtype=jnp.int32) % (NPAD - N_NODES)
    src3 = jnp.concatenate([src, pad_src]).reshape(NW, NCHUNK, ECHUNK)
    dst3 = jnp.concatenate([dst, pad_dst]).reshape(NW, NCHUNK, ECHUNK)

    h, degp = _sc_gather_deg(xp, dst3, emb)
    hws1, dinv = _tc_prep1(degp.T, h, W1)
    accp1 = _sc_edge_agg(hws1, src3, dst3)
    hws2 = _tc_combine1(accp1, hws1, dinv, b1, g1, be1, W2)
    accp2 = _sc_edge_agg(hws2, src3, dst3)
    out_col = _tc_combine2(accp2, hws2, dinv, b2, g2, be2, batch_col, fcW, fcb)
    return out_col[:, 0]


# speedup vs baseline: 27.1697x; 1.0430x over previous
import functools

import jax
import jax.numpy as jnp
from jax import lax
from jax.experimental import pallas as pl
from jax.experimental.pallas import tpu as pltpu
import jax.experimental.pallas.tpu_sc as plsc

N_NODES = 10000
N_EDGES = 320000
N_GRAPHS = 64
D = 128

NC, NS = 2, 16
NW = NC * NS
NPAD = 10240
ROWS_PW = NPAD // NW
ECHUNK = 128
NCHUNK = 80
EPAD = NW * NCHUNK * ECHUNK
EDGES_PW = NCHUNK * ECHUNK
GCHUNK = 80
ROWS_PT = NPAD // NS

_BLK = 1024
_NBLK = NPAD // _BLK


def _mesh():
    return plsc.VectorSubcoreMesh(core_axis_name="c", subcore_axis_name="s")


def _sc_gather_deg(xp, dst3, emb):
    @functools.partial(
        pl.kernel,
        mesh=_mesh(),
        out_type=[
            jax.ShapeDtypeStruct((NPAD, D), jnp.float32),
            jax.ShapeDtypeStruct((NC, NPAD), jnp.float32),
        ],
        scratch_types=[
            pltpu.VMEM((ROWS_PW,), jnp.int32),
            pltpu.VMEM((NCHUNK, ECHUNK), jnp.int32),
            pltpu.VMEM((ECHUNK,), jnp.float32),
            pltpu.VMEM((ROWS_PT,), jnp.float32),
            pltpu.VMEM((GCHUNK, D), jnp.float32),
            pltpu.VMEM((GCHUNK, D), jnp.float32),
            pltpu.VMEM_SHARED((NPAD,), jnp.float32),
            pltpu.SemaphoreType.DMA,
            pltpu.SemaphoreType.DMA,
        ],
    )
    def body(x_hbm, dst_hbm, emb_hbm, h_out, deg_out, idx4, dst2d,
             ones_v, zbuf, rows_a, rows_b, shared_deg, sem_g, sem_h):
        c = lax.axis_index("c")
        s = lax.axis_index("s")
        wid = s * NC + c

        for j in range(ECHUNK // 16):
            ones_v[pl.ds(16 * j, 16)] = jnp.ones((16,), jnp.float32)
        for j in range(ROWS_PT // 16):
            zbuf[pl.ds(16 * j, 16)] = jnp.zeros((16,), jnp.float32)

        pltpu.sync_copy(zbuf, shared_deg.at[pl.ds(s * ROWS_PT, ROWS_PT)])
        pltpu.sync_copy(dst_hbm.at[wid], dst2d)
        plsc.subcore_barrier()

        nge = ROWS_PW // GCHUNK
        pltpu.sync_copy(x_hbm.at[pl.ds(wid * ROWS_PW, ROWS_PW)], idx4)
        ids = [idx4.at[pl.ds(j * GCHUNK, GCHUNK)] for j in range(nge)]
        rows = [rows_a, rows_b]
        pltpu.async_copy(emb_hbm.at[ids[0]], rows_a, sem_g)
        for j in range(nge):
            pltpu.make_async_copy(emb_hbm.at[ids[j]], rows[j % 2],
                                  sem_g).wait()
            if j + 1 < nge:
                pltpu.async_copy(emb_hbm.at[ids[j + 1]],
                                 rows[(j + 1) % 2], sem_g)
            pltpu.sync_copy(rows[j % 2],
                            h_out.at[pl.ds(wid * ROWS_PW + j * GCHUNK,
                                           GCHUNK)])

        def hgroup(g, carry):
            for j in range(8):
                pltpu.async_copy(ones_v, shared_deg.at[dst2d.at[g * 8 + j]],
                                 sem_h, add=True)
            for j in range(8):
                pltpu.make_async_copy(ones_v,
                                      shared_deg.at[dst2d.at[g * 8 + j]],
                                      sem_h).wait()
            return carry

        lax.fori_loop(0, NCHUNK // 8, hgroup, 0)
        plsc.subcore_barrier()

        pltpu.sync_copy(shared_deg.at[pl.ds(s * ROWS_PT, ROWS_PT)], zbuf)
        pltpu.sync_copy(zbuf, deg_out.at[c, pl.ds(s * ROWS_PT, ROWS_PT)])

    return body(xp, dst3, emb)


def _sc_edge_agg(hws, src3, dst3):
    @functools.partial(
        pl.kernel,
        mesh=_mesh(),
        out_type=jax.ShapeDtypeStruct((NC, NPAD, D), jnp.float32),
        scratch_types=[
            pltpu.VMEM((2, ECHUNK), jnp.int32),
            pltpu.VMEM((NCHUNK, ECHUNK), jnp.int32),
            pltpu.VMEM((ECHUNK, D), jnp.float32),
            pltpu.VMEM((ECHUNK, D), jnp.float32),
            pltpu.VMEM_SHARED((NPAD, D), jnp.float32),
            pltpu.SemaphoreType.DMA,
            pltpu.SemaphoreType.DMA,
            pltpu.SemaphoreType.DMA,
        ],
    )
    def body(hws_hbm, src_hbm, dst_hbm, acc_out, src_sm, dst2d, rows_a,
             rows_b, shared_acc, sem_g, sem_i, sem_f):
        c = lax.axis_index("c")
        s = lax.axis_index("s")
        wid = s * NC + c
        rows = [rows_a, rows_b]

        pltpu.sync_copy(dst_hbm.at[wid], dst2d)

        def zb(i, carry):
            for j in range(8):
                rows_a[i, pl.ds(j * 16, 16)] = jnp.zeros((16,), jnp.float32)
            return carry
        lax.fori_loop(0, ECHUNK, zb, 0)
        nz = -(-ROWS_PT // ECHUNK)
        for k in range(nz):
            r0 = s * ROWS_PT + k * ECHUNK
            n = min(ECHUNK, ROWS_PT - k * ECHUNK)
            pltpu.sync_copy(rows_a.at[pl.ds(0, n)],
                            shared_acc.at[pl.ds(r0, n)])
        plsc.subcore_barrier()

        pltpu.sync_copy(src_hbm.at[wid, 0], src_sm.at[0])
        pltpu.async_copy(hws_hbm.at[src_sm.at[0]], rows_a, sem_g)
        pltpu.async_copy(src_hbm.at[wid, 1], src_sm.at[1], sem_i)

        def pair(k, carry):
            i0 = 2 * k
            for j in range(2):
                i = i0 + j
                pltpu.make_async_copy(hws_hbm.at[src_sm.at[j % 2]],
                                      rows[j % 2], sem_g).wait()

                @pl.when(i + 1 < NCHUNK)
                def _():
                    pltpu.make_async_copy(src_hbm.at[wid, i + 1],
                                          src_sm.at[(j + 1) % 2], sem_i).wait()
                    pltpu.async_copy(hws_hbm.at[src_sm.at[(j + 1) % 2]],
                                     rows[(j + 1) % 2], sem_g)

                @pl.when(i + 2 < NCHUNK)
                def _():
                    pltpu.async_copy(src_hbm.at[wid, i + 2],
                                     src_sm.at[j % 2], sem_i)

                pltpu.sync_copy(rows[j % 2],
                                shared_acc.at[dst2d.at[i]],
                                add=True)
            return carry

        lax.fori_loop(0, NCHUNK // 2, pair, 0)
        plsc.subcore_barrier()

        FCH = 80
        nf = ROWS_PT // FCH
        for k in range(nf):
            r0 = s * ROWS_PT + k * FCH
            buf = rows[k % 2].at[pl.ds(0, FCH)]
            if k >= 2:
                rp = s * ROWS_PT + (k - 2) * FCH
                pltpu.make_async_copy(rows[k % 2].at[pl.ds(0, FCH)],
                                      acc_out.at[c, pl.ds(rp, FCH)],
                                      sem_f).wait()
            pltpu.sync_copy(shared_acc.at[pl.ds(r0, FCH)], buf)
            pltpu.async_copy(buf, acc_out.at[c, pl.ds(r0, FCH)], sem_f)
        for k in range(nf - 2, nf):
            r0 = s * ROWS_PT + k * FCH
            pltpu.make_async_copy(rows[k % 2].at[pl.ds(0, FCH)],
                                  acc_out.at[c, pl.ds(r0, FCH)],
                                  sem_f).wait()

    return body(hws, src3, dst3)


def _tc_prep1(deg_t, h, W1):
    def body(degt_ref, h_ref, w_ref, hws_ref, dinv_ref):
        deg = degt_ref[:, 0:1] + degt_ref[:, 1:2] + 1.0
        dinv = lax.rsqrt(deg)
        dinv_ref[...] = dinv
        hw = jnp.dot(h_ref[...], w_ref[...], preferred_element_type=jnp.float32)
        hws_ref[...] = hw * dinv

    return pl.pallas_call(
        body,
        grid=(_NBLK,),
        in_specs=[
            pl.BlockSpec((_BLK, 2), lambda i: (i, 0)),
            pl.BlockSpec((_BLK, D), lambda i: (i, 0)),
            pl.BlockSpec((D, D), lambda i: (0, 0)),
        ],
        out_specs=[
            pl.BlockSpec((_BLK, D), lambda i: (i, 0)),
            pl.BlockSpec((_BLK, 1), lambda i: (i, 0)),
        ],
        out_shape=[
            jax.ShapeDtypeStruct((NPAD, D), jnp.float32),
            jax.ShapeDtypeStruct((NPAD, 1), jnp.float32),
        ],
    )(deg_t, h, W1)


def _norm_act(accp, hws, dinv, b, g, be):
    acc = accp[0] + accp[1]
    pre = dinv * (acc + hws) + b
    mu = jnp.mean(pre, axis=1, keepdims=True)
    yc = pre - mu
    var = jnp.mean(yc * yc, axis=1, keepdims=True)
    hn = yc * lax.rsqrt(var + 1e-5) * g + be
    return jnp.maximum(hn, 0.0)


def _tc_combine1(accp, hws1, dinv, b1, g1, be1, W2):
    def body(accp_ref, hws_ref, dinv_ref, b_ref, g_ref, be_ref, w_ref, out_ref):
        h1 = _norm_act(accp_ref[...], hws_ref[...], dinv_ref[...],
                       b_ref[...], g_ref[...], be_ref[...])
        hw = jnp.dot(h1, w_ref[...], preferred_element_type=jnp.float32)
        out_ref[...] = hw * dinv_ref[...]

    return pl.pallas_call(
        body,
        grid=(_NBLK,),
        in_specs=[
            pl.BlockSpec((NC, _BLK, D), lambda i: (0, i, 0)),
            pl.BlockSpec((_BLK, D), lambda i: (i, 0)),
            pl.BlockSpec((_BLK, 1), lambda i: (i, 0)),
            pl.BlockSpec((1, D), lambda i: (0, 0)),
            pl.BlockSpec((1, D), lambda i: (0, 0)),
            pl.BlockSpec((1, D), lambda i: (0, 0)),
            pl.BlockSpec((D, D), lambda i: (0, 0)),
        ],
        out_specs=pl.BlockSpec((_BLK, D), lambda i: (i, 0)),
        out_shape=jax.ShapeDtypeStruct((NPAD, D), jnp.float32),
    )(accp, hws1, dinv, b1.reshape(1, D), g1.reshape(1, D),
      be1.reshape(1, D), W2)


def _tc_combine2(accp, hws2, dinv, b2, g2, be2, batch_col, fcW, fcb):
    def body(accp_ref, hws_ref, dinv_ref, b_ref, g_ref, be_ref, bat_ref,
             fcw_ref, fcb_ref, out_ref, sums_sc, cnts_sc):
        h2 = _norm_act(accp_ref[...], hws_ref[...], dinv_ref[...],
                       b_ref[...], g_ref[...], be_ref[...])
        bid = bat_ref[...]
        gi = lax.broadcasted_iota(jnp.int32, (_BLK, N_GRAPHS), 1)
        oh = (bid == gi).astype(jnp.float32)
        dn = (((0,), (0,)), ((), ()))
        ps = lax.dot_general(oh, h2, dn, preferred_element_type=jnp.float32)
        ones_m = jnp.ones((_BLK, D), jnp.float32)
        pc = lax.dot_general(oh, ones_m, dn, preferred_element_type=jnp.float32)

        i = pl.program_id(0)

        @pl.when(i == 0)
        def _():
            sums_sc[...] = ps
            cnts_sc[...] = pc
            out_ref[...] = jnp.zeros((N_GRAPHS, 1), jnp.float32)

        @pl.when(i > 0)
        def _():
            sums_sc[...] += ps
            cnts_sc[...] += pc

        @pl.when(i == _NBLK - 1)
        def _():
            pooled = sums_sc[...] / jnp.maximum(cnts_sc[...], 1.0)
            out_ref[...] = (jnp.dot(pooled, fcw_ref[...],
                                    preferred_element_type=jnp.float32)
                            + fcb_ref[...])

    return pl.pallas_call(
        body,
        grid=(_NBLK,),
        in_specs=[
            pl.BlockSpec((NC, _BLK, D), lambda i: (0, i, 0)),
            pl.BlockSpec((_BLK, D), lambda i: (i, 0)),
            pl.BlockSpec((_BLK, 1), lambda i: (i, 0)),
            pl.BlockSpec((1, D), lambda i: (0, 0)),
            pl.BlockSpec((1, D), lambda i: (0, 0)),
            pl.BlockSpec((1, D), lambda i: (0, 0)),
            pl.BlockSpec((_BLK, 1), lambda i: (i, 0)),
            pl.BlockSpec((D, 1), lambda i: (0, 0)),
            pl.BlockSpec((1, 1), lambda i: (0, 0)),
        ],
        out_specs=pl.BlockSpec((N_GRAPHS, 1), lambda i: (0, 0)),
        out_shape=jax.ShapeDtypeStruct((N_GRAPHS, 1), jnp.float32),
        scratch_shapes=[
            pltpu.VMEM((N_GRAPHS, D), jnp.float32),
            pltpu.VMEM((N_GRAPHS, D), jnp.float32),
        ],
    )(accp, hws2, dinv, b2.reshape(1, D), g2.reshape(1, D), be2.reshape(1, D),
      batch_col, fcW, fcb.reshape(1, 1))


def kernel(x, edge_index, batch, emb, W1, b1, g1, be1, W2, b2, g2, be2,
           fcW, fcb):
    x = x.astype(jnp.int32)
    src = edge_index[0].astype(jnp.int32)
    dst = edge_index[1].astype(jnp.int32)
    xp = jnp.concatenate([x, jnp.arange(NPAD - N_NODES, dtype=jnp.int32)])
    batch_col = jnp.concatenate(
        [batch.astype(jnp.int32),
         jnp.full((NPAD - N_NODES,), N_GRAPHS, jnp.int32)]).reshape(NPAD, 1)
    npe = EPAD - N_EDGES
    pad_src = jnp.arange(npe, dtype=jnp.int32) % N_NODES
    pad_dst = N_NODES + jnp.arange(npe, dtype=jnp.int32) % (NPAD - N_NODES)
    src3 = jnp.concatenate([src, pad_src]).reshape(NW, NCHUNK, ECHUNK)
    dst3 = jnp.concatenate([dst, pad_dst]).reshape(NW, NCHUNK, ECHUNK)

    h, degp = _sc_gather_deg(xp, dst3, emb)
    hws1, dinv = _tc_prep1(degp.T, h, W1)
    accp1 = _sc_edge_agg(hws1, src3, dst3)
    hws2 = _tc_combine1(accp1, hws1, dinv, b1, g1, be1, W2)
    accp2 = _sc_edge_agg(hws2, src3, dst3)
    out_col = _tc_combine2(accp2, hws2, dinv, b2, g2, be2, batch_col, fcW, fcb)
    return out_col[:, 0]


# speedup vs baseline: 33.6730x; 1.2394x over previous
import functools

import jax
import jax.numpy as jnp
from jax import lax
from jax.experimental import pallas as pl
from jax.experimental.pallas import tpu as pltpu
import jax.experimental.pallas.tpu_sc as plsc

N_NODES = 10000
N_EDGES = 320000
N_GRAPHS = 64
D = 128

NC, NS = 2, 16
NW = NC * NS
NPAD = 10240
ROWS_PW = NPAD // NW
ECHUNK = 128
NCHUNK = 81
EPAD = NW * NCHUNK * ECHUNK
EDGES_PW = NCHUNK * ECHUNK
GCHUNK = 80
ROWS_PT = NPAD // NS
ACC_ROWS = 10112
APT = ACC_ROWS // NS

_BLK = 1024
_NBLK = NPAD // _BLK


def _mesh():
    return plsc.VectorSubcoreMesh(core_axis_name="c", subcore_axis_name="s")


def _sc_gather_deg(xp, dst3, emb):
    @functools.partial(
        pl.kernel,
        mesh=_mesh(),
        out_type=[
            jax.ShapeDtypeStruct((NPAD, D), jnp.float32),
            jax.ShapeDtypeStruct((NC, NPAD), jnp.float32),
        ],
        scratch_types=[
            pltpu.VMEM((ROWS_PW,), jnp.int32),
            pltpu.VMEM((NCHUNK, ECHUNK), jnp.int32),
            pltpu.VMEM((ECHUNK,), jnp.float32),
            pltpu.VMEM((ROWS_PT,), jnp.float32),
            pltpu.VMEM((GCHUNK, D), jnp.float32),
            pltpu.VMEM((GCHUNK, D), jnp.float32),
            pltpu.VMEM_SHARED((NPAD,), jnp.float32),
            pltpu.SemaphoreType.DMA,
            pltpu.SemaphoreType.DMA,
        ],
    )
    def body(x_hbm, dst_hbm, emb_hbm, h_out, deg_out, idx4, dst2d,
             ones_v, zbuf, rows_a, rows_b, shared_deg, sem_g, sem_h):
        c = lax.axis_index("c")
        s = lax.axis_index("s")
        wid = s * NC + c

        for j in range(ECHUNK // 16):
            ones_v[pl.ds(16 * j, 16)] = jnp.ones((16,), jnp.float32)
        for j in range(ROWS_PT // 16):
            zbuf[pl.ds(16 * j, 16)] = jnp.zeros((16,), jnp.float32)

        pltpu.sync_copy(zbuf, shared_deg.at[pl.ds(s * ROWS_PT, ROWS_PT)])
        pltpu.sync_copy(dst_hbm.at[wid], dst2d)
        plsc.subcore_barrier()

        nge = ROWS_PW // GCHUNK
        pltpu.sync_copy(x_hbm.at[pl.ds(wid * ROWS_PW, ROWS_PW)], idx4)
        ids = [idx4.at[pl.ds(j * GCHUNK, GCHUNK)] for j in range(nge)]
        rows = [rows_a, rows_b]
        pltpu.async_copy(emb_hbm.at[ids[0]], rows_a, sem_g)
        for j in range(nge):
            pltpu.make_async_copy(emb_hbm.at[ids[j]], rows[j % 2],
                                  sem_g).wait()
            if j + 1 < nge:
                pltpu.async_copy(emb_hbm.at[ids[j + 1]],
                                 rows[(j + 1) % 2], sem_g)
            pltpu.sync_copy(rows[j % 2],
                            h_out.at[pl.ds(wid * ROWS_PW + j * GCHUNK,
                                           GCHUNK)])

        def hgroup(g, carry):
            for j in range(8):
                pltpu.async_copy(ones_v, shared_deg.at[dst2d.at[g * 8 + j]],
                                 sem_h, add=True)
            for j in range(8):
                pltpu.make_async_copy(ones_v,
                                      shared_deg.at[dst2d.at[g * 8 + j]],
                                      sem_h).wait()
            return carry

        lax.fori_loop(0, NCHUNK // 8, hgroup, 0)
        for t in range(NCHUNK - 8 * (NCHUNK // 8)):
            i = 8 * (NCHUNK // 8) + t
            pltpu.async_copy(ones_v, shared_deg.at[dst2d.at[i]],
                             sem_h, add=True)
            pltpu.make_async_copy(ones_v, shared_deg.at[dst2d.at[i]],
                                  sem_h).wait()
        plsc.subcore_barrier()

        pltpu.sync_copy(shared_deg.at[pl.ds(s * ROWS_PT, ROWS_PT)], zbuf)
        pltpu.sync_copy(zbuf, deg_out.at[c, pl.ds(s * ROWS_PT, ROWS_PT)])

    return body(xp, dst3, emb)


def _sc_edge_agg(hws, src3, dst3):
    @functools.partial(
        pl.kernel,
        mesh=_mesh(),
        out_type=jax.ShapeDtypeStruct((NC, NPAD, D), jnp.float32),
        scratch_types=[
            pltpu.VMEM((4, ECHUNK), jnp.int32),
            pltpu.VMEM((4, ECHUNK), jnp.int32),
            pltpu.VMEM((ECHUNK, D), jnp.float32),
            pltpu.VMEM((ECHUNK, D), jnp.float32),
            pltpu.VMEM((ECHUNK, D), jnp.float32),
            pltpu.VMEM_SHARED((ACC_ROWS, D), jnp.float32),
            pltpu.SemaphoreType.DMA,
            pltpu.SemaphoreType.DMA,
            pltpu.SemaphoreType.DMA,
            pltpu.SemaphoreType.DMA,
            pltpu.SemaphoreType.DMA,
        ],
    )
    def body(hws_hbm, src_hbm, dst_hbm, acc_out, src_sm, dst_sm, rows_a,
             rows_b, rows_c, shared_acc, sem_a, sem_b, sem_c, sem_i, sem_f):
        c = lax.axis_index("c")
        s = lax.axis_index("s")
        wid = s * NC + c
        rows = [rows_a, rows_b, rows_c]
        gsem = [sem_a, sem_b, sem_c]

        def zb(i, carry):
            for j in range(8):
                rows_a[i, pl.ds(j * 16, 16)] = jnp.zeros((16,), jnp.float32)
            return carry
        lax.fori_loop(0, ECHUNK, zb, 0)
        nz = -(-APT // ECHUNK)
        for k in range(nz):
            r0 = s * APT + k * ECHUNK
            n = min(ECHUNK, APT - k * ECHUNK)
            pltpu.sync_copy(rows_a.at[pl.ds(0, n)],
                            shared_acc.at[pl.ds(r0, n)])

        @pl.when(s == 0)
        def _():
            pltpu.sync_copy(rows_a, acc_out.at[c, pl.ds(ACC_ROWS, ECHUNK)])
        plsc.subcore_barrier()

        pltpu.sync_copy(src_hbm.at[wid, 0], src_sm.at[0])
        pltpu.sync_copy(dst_hbm.at[wid, 0], dst_sm.at[0])
        pltpu.sync_copy(src_hbm.at[wid, 1], src_sm.at[1])
        pltpu.sync_copy(dst_hbm.at[wid, 1], dst_sm.at[1])
        pltpu.async_copy(src_hbm.at[wid, 2], src_sm.at[2], sem_i)
        pltpu.async_copy(dst_hbm.at[wid, 2], dst_sm.at[2], sem_i)
        pltpu.async_copy(hws_hbm.at[src_sm.at[0]], rows_a, sem_a)
        pltpu.async_copy(hws_hbm.at[src_sm.at[1]], rows_b, sem_b)

        def tri(k, carry):
            for j in range(3):
                i = 3 * k + j
                sl = lax.rem(i, 4)
                pltpu.make_async_copy(hws_hbm.at[src_sm.at[sl]],
                                      rows[j], gsem[j]).wait()

                @pl.when(i + 2 < NCHUNK)
                def _():
                    sl2 = lax.rem(i + 2, 4)
                    pltpu.make_async_copy(src_hbm.at[wid, i + 2],
                                          src_sm.at[sl2], sem_i).wait()
                    pltpu.make_async_copy(dst_hbm.at[wid, i + 2],
                                          dst_sm.at[sl2], sem_i).wait()
                    pltpu.async_copy(hws_hbm.at[src_sm.at[sl2]],
                                     rows[(j + 2) % 3], gsem[(j + 2) % 3])

                @pl.when(i + 3 < NCHUNK)
                def _():
                    sl3 = lax.rem(i + 3, 4)
                    pltpu.async_copy(src_hbm.at[wid, i + 3],
                                     src_sm.at[sl3], sem_i)
                    pltpu.async_copy(dst_hbm.at[wid, i + 3],
                                     dst_sm.at[sl3], sem_i)

                pltpu.sync_copy(rows[j], shared_acc.at[dst_sm.at[sl]],
                                add=True)
            return carry

        lax.fori_loop(0, NCHUNK // 3, tri, 0)
        plsc.subcore_barrier()

        nf = -(-APT // ECHUNK)
        sizes = [min(ECHUNK, APT - k * ECHUNK) for k in range(nf)]
        for k in range(nf):
            r0 = s * APT + k * ECHUNK
            buf = rows[k % 2].at[pl.ds(0, sizes[k])]
            if k >= 2:
                rp = s * APT + (k - 2) * ECHUNK
                pltpu.make_async_copy(rows[k % 2].at[pl.ds(0, sizes[k - 2])],
                                      acc_out.at[c, pl.ds(rp, sizes[k - 2])],
                                      sem_f).wait()
            pltpu.sync_copy(shared_acc.at[pl.ds(r0, sizes[k])], buf)
            pltpu.async_copy(buf, acc_out.at[c, pl.ds(r0, sizes[k])], sem_f)
        for k in range(nf - 2, nf):
            r0 = s * APT + k * ECHUNK
            pltpu.make_async_copy(rows[k % 2].at[pl.ds(0, sizes[k])],
                                  acc_out.at[c, pl.ds(r0, sizes[k])],
                                  sem_f).wait()

    return body(hws, src3, dst3)


def _tc_prep1(deg_t, h, W1):
    def body(degt_ref, h_ref, w_ref, hws_ref, dinv_ref):
        deg = degt_ref[:, 0:1] + degt_ref[:, 1:2] + 1.0
        dinv = lax.rsqrt(deg)
        dinv_ref[...] = dinv
        hw = jnp.dot(h_ref[...], w_ref[...], preferred_element_type=jnp.float32)
        hws_ref[...] = hw * dinv

    return pl.pallas_call(
        body,
        grid=(_NBLK,),
        in_specs=[
            pl.BlockSpec((_BLK, 2), lambda i: (i, 0)),
            pl.BlockSpec((_BLK, D), lambda i: (i, 0)),
            pl.BlockSpec((D, D), lambda i: (0, 0)),
        ],
        out_specs=[
            pl.BlockSpec((_BLK, D), lambda i: (i, 0)),
            pl.BlockSpec((_BLK, 1), lambda i: (i, 0)),
        ],
        out_shape=[
            jax.ShapeDtypeStruct((NPAD, D), jnp.float32),
            jax.ShapeDtypeStruct((NPAD, 1), jnp.float32),
        ],
    )(deg_t, h, W1)


def _norm_act(accp, hws, dinv, b, g, be):
    acc = accp[0] + accp[1]
    pre = dinv * (acc + hws) + b
    mu = jnp.mean(pre, axis=1, keepdims=True)
    yc = pre - mu
    var = jnp.mean(yc * yc, axis=1, keepdims=True)
    hn = yc * lax.rsqrt(var + 1e-5) * g + be
    return jnp.maximum(hn, 0.0)


def _tc_combine1(accp, hws1, dinv, b1, g1, be1, W2):
    def body(accp_ref, hws_ref, dinv_ref, b_ref, g_ref, be_ref, w_ref, out_ref):
        h1 = _norm_act(accp_ref[...], hws_ref[...], dinv_ref[...],
                       b_ref[...], g_ref[...], be_ref[...])
        hw = jnp.dot(h1, w_ref[...], preferred_element_type=jnp.float32)
        out_ref[...] = hw * dinv_ref[...]

    return pl.pallas_call(
        body,
        grid=(_NBLK,),
        in_specs=[
            pl.BlockSpec((NC, _BLK, D), lambda i: (0, i, 0)),
            pl.BlockSpec((_BLK, D), lambda i: (i, 0)),
            pl.BlockSpec((_BLK, 1), lambda i: (i, 0)),
            pl.BlockSpec((1, D), lambda i: (0, 0)),
            pl.BlockSpec((1, D), lambda i: (0, 0)),
            pl.BlockSpec((1, D), lambda i: (0, 0)),
            pl.BlockSpec((D, D), lambda i: (0, 0)),
        ],
        out_specs=pl.BlockSpec((_BLK, D), lambda i: (i, 0)),
        out_shape=jax.ShapeDtypeStruct((NPAD, D), jnp.float32),
    )(accp, hws1, dinv, b1.reshape(1, D), g1.reshape(1, D),
      be1.reshape(1, D), W2)


def _tc_combine2(accp, hws2, dinv, b2, g2, be2, batch_col, fcW, fcb):
    def body(accp_ref, hws_ref, dinv_ref, b_ref, g_ref, be_ref, bat_ref,
             fcw_ref, fcb_ref, out_ref, sums_sc, cnts_sc):
        h2 = _norm_act(accp_ref[...], hws_ref[...], dinv_ref[...],
                       b_ref[...], g_ref[...], be_ref[...])
        bid = bat_ref[...]
        gi = lax.broadcasted_iota(jnp.int32, (_BLK, N_GRAPHS), 1)
        oh = (bid == gi).astype(jnp.float32)
        dn = (((0,), (0,)), ((), ()))
        ps = lax.dot_general(oh, h2, dn, preferred_element_type=jnp.float32)
        ones_m = jnp.ones((_BLK, D), jnp.float32)
        pc = lax.dot_general(oh, ones_m, dn, preferred_element_type=jnp.float32)

        i = pl.program_id(0)

        @pl.when(i == 0)
        def _():
            sums_sc[...] = ps
            cnts_sc[...] = pc
            out_ref[...] = jnp.zeros((N_GRAPHS, 1), jnp.float32)

        @pl.when(i > 0)
        def _():
            sums_sc[...] += ps
            cnts_sc[...] += pc

        @pl.when(i == _NBLK - 1)
        def _():
            pooled = sums_sc[...] / jnp.maximum(cnts_sc[...], 1.0)
            out_ref[...] = (jnp.dot(pooled, fcw_ref[...],
                                    preferred_element_type=jnp.float32)
                            + fcb_ref[...])

    return pl.pallas_call(
        body,
        grid=(_NBLK,),
        in_specs=[
            pl.BlockSpec((NC, _BLK, D), lambda i: (0, i, 0)),
            pl.BlockSpec((_BLK, D), lambda i: (i, 0)),
            pl.BlockSpec((_BLK, 1), lambda i: (i, 0)),
            pl.BlockSpec((1, D), lambda i: (0, 0)),
            pl.BlockSpec((1, D), lambda i: (0, 0)),
            pl.BlockSpec((1, D), lambda i: (0, 0)),
            pl.BlockSpec((_BLK, 1), lambda i: (i, 0)),
            pl.BlockSpec((D, 1), lambda i: (0, 0)),
            pl.BlockSpec((1, 1), lambda i: (0, 0)),
        ],
        out_specs=pl.BlockSpec((N_GRAPHS, 1), lambda i: (0, 0)),
        out_shape=jax.ShapeDtypeStruct((N_GRAPHS, 1), jnp.float32),
        scratch_shapes=[
            pltpu.VMEM((N_GRAPHS, D), jnp.float32),
            pltpu.VMEM((N_GRAPHS, D), jnp.float32),
        ],
    )(accp, hws2, dinv, b2.reshape(1, D), g2.reshape(1, D), be2.reshape(1, D),
      batch_col, fcW, fcb.reshape(1, 1))


def kernel(x, edge_index, batch, emb, W1, b1, g1, be1, W2, b2, g2, be2,
           fcW, fcb):
    x = x.astype(jnp.int32)
    src = edge_index[0].astype(jnp.int32)
    dst = edge_index[1].astype(jnp.int32)
    xp = jnp.concatenate([x, jnp.arange(NPAD - N_NODES, dtype=jnp.int32)])
    batch_col = jnp.concatenate(
        [batch.astype(jnp.int32),
         jnp.full((NPAD - N_NODES,), N_GRAPHS, jnp.int32)]).reshape(NPAD, 1)
    npe = EPAD - N_EDGES
    pad_src = jnp.arange(npe, dtype=jnp.int32) % N_NODES
    pad_dst = N_NODES + jnp.arange(npe, dtype=jnp.int32) % (ACC_ROWS - N_NODES)
    src3 = jnp.concatenate([src, pad_src]).reshape(NW, NCHUNK, ECHUNK)
    dst3 = jnp.concatenate([dst, pad_dst]).reshape(NW, NCHUNK, ECHUNK)

    h, degp = _sc_gather_deg(xp, dst3, emb)
    hws1, dinv = _tc_prep1(degp.T, h, W1)
    accp1 = _sc_edge_agg(hws1, src3, dst3)
    hws2 = _tc_combine1(accp1, hws1, dinv, b1, g1, be1, W2)
    accp2 = _sc_edge_agg(hws2, src3, dst3)
    out_col = _tc_combine2(accp2, hws2, dinv, b2, g2, be2, batch_col, fcW, fcb)
    return out_col[:, 0]
